# speedup vs baseline: 2.7527x; 2.7527x over previous
import functools

import jax
import jax.numpy as jnp
from jax import lax
from jax.experimental import pallas as pl
from jax.experimental.pallas import tpu as pltpu
from jax.experimental.pallas import tpu_sc as plsc

B = 4096 * 50
D = 128

_info = plsc.get_sparse_core_info()
NC, NS = _info.num_cores, _info.num_subcores
NW = NC * NS
B_PER_W = B // NW
CHUNK = 128
N_CHUNKS = B_PER_W // CHUNK

_mesh = plsc.VectorSubcoreMesh(core_axis_name="c", subcore_axis_name="s")


@functools.partial(
    pl.kernel,
    mesh=_mesh,
    out_type=jax.ShapeDtypeStruct((B, D), jnp.float32),
    scratch_types=[
        pltpu.VMEM((CHUNK,), jnp.int32),
        pltpu.VMEM((CHUNK, D), jnp.float32),
        pltpu.SemaphoreType.DMA,
    ],
)
def _gather_kernel(idx_hbm, table_hbm, out_hbm, idx_v, rows_v, sem):
    wid = lax.axis_index("s") * NC + lax.axis_index("c")
    base = wid * B_PER_W

    def body(i, _):
        off = base + i * CHUNK
        pltpu.sync_copy(idx_hbm.at[pl.ds(off, CHUNK)], idx_v)
        pltpu.async_copy(table_hbm.at[idx_v], rows_v, sem).wait()
        pltpu.sync_copy(rows_v, out_hbm.at[pl.ds(off, CHUNK)])
        return 0

    lax.fori_loop(0, N_CHUNKS, body, 0)


def kernel(X, W):
    idx = X.reshape(-1).astype(jnp.int32)
    out = _gather_kernel(idx, W)
    return out.reshape(X.shape + (W.shape[-1],))


# speedup vs baseline: 3.2999x; 1.1988x over previous
import functools

import jax
import jax.numpy as jnp
from jax import lax
from jax.experimental import pallas as pl
from jax.experimental.pallas import tpu as pltpu
from jax.experimental.pallas import tpu_sc as plsc

B = 4096 * 50
D = 128

_info = plsc.get_sparse_core_info()
NC, NS = _info.num_cores, _info.num_subcores
NW = NC * NS
B_PER_W = B // NW
CHUNK = 128
N_CHUNKS = B_PER_W // CHUNK
NBUF = 5
N_GROUPS = N_CHUNKS // NBUF

_mesh = plsc.VectorSubcoreMesh(core_axis_name="c", subcore_axis_name="s")


@functools.partial(
    pl.kernel,
    mesh=_mesh,
    out_type=jax.ShapeDtypeStruct((B, D), jnp.float32),
    scratch_types=[
        pltpu.VMEM((1, N_CHUNKS, CHUNK), jnp.int32),
        pltpu.VMEM((NBUF, CHUNK, D), jnp.float32),
    ]
    + [pltpu.SemaphoreType.DMA] * (2 * NBUF),
)
def _gather_kernel(idx_hbm, table_hbm, out_hbm, idx_v, rows_v, *sems):
    gsem = sems[:NBUF]
    osem = sems[NBUF:]
    wid = lax.axis_index("s") * NC + lax.axis_index("c")
    cbase = wid * N_CHUNKS

    pltpu.sync_copy(idx_hbm.at[pl.ds(wid, 1)], idx_v)

    def gather_desc(chunk, b):
        return pltpu.make_async_copy(
            table_hbm.at[idx_v.at[0, chunk]], rows_v.at[b], gsem[b]
        )

    def out_desc(chunk, b):
        row0 = (cbase + chunk) * CHUNK
        return pltpu.make_async_copy(
            rows_v.at[b], out_hbm.at[pl.ds(row0, CHUNK)], osem[b]
        )

    for b in range(NBUF):
        gather_desc(b, b).start()

    def body(g, _):
        for b in range(NBUF):
            chunk = g * NBUF + b
            gather_desc(chunk, b).wait()
            out_desc(chunk, b).start()
        for b in range(NBUF):
            chunk = g * NBUF + b
            out_desc(chunk, b).wait()

            @pl.when(g + 1 < N_GROUPS)
            def _():
                gather_desc(chunk + NBUF, b).start()

        return 0

    lax.fori_loop(0, N_GROUPS, body, 0)


def kernel(X, W):
    idx = X.reshape(NW, N_CHUNKS, CHUNK).astype(jnp.int32)
    out = _gather_kernel(idx, W)
    return out.reshape(X.shape + (W.shape[-1],))


# speedup vs baseline: 5.9489x; 1.8027x over previous
import functools

import jax
import jax.numpy as jnp
from jax import lax
from jax.experimental import pallas as pl
from jax.experimental.pallas import tpu as pltpu
from jax.experimental.pallas import tpu_sc as plsc

NBATCH = 4096
SEQ = 50
D = 128

_info = plsc.get_sparse_core_info()
NC, NS = _info.num_cores, _info.num_subcores
NW = NC * NS
BAT_PER_W = NBATCH // NW
NB = 8
NSLAB = BAT_PER_W // NB
N_PAIRS = NSLAB // 2

_mesh = plsc.VectorSubcoreMesh(core_axis_name="c", subcore_axis_name="s")


@functools.partial(
    pl.kernel,
    mesh=_mesh,
    out_type=jax.ShapeDtypeStruct((NBATCH, SEQ, D), jnp.float32),
    scratch_types=[
        pltpu.VMEM((BAT_PER_W, SEQ), jnp.int32),
        pltpu.VMEM((2, NB, SEQ, D), jnp.float32),
        pltpu.SemaphoreType.DMA,
        pltpu.SemaphoreType.DMA,
        pltpu.SemaphoreType.DMA,
        pltpu.SemaphoreType.DMA,
    ],
)
def _gather_kernel(idx_hbm, table_hbm, out_hbm, idx_v, rows_v, g0, g1, o0, o1):
    gsem = (g0, g1)
    osem = (o0, o1)
    wid = lax.axis_index("s") * NC + lax.axis_index("c")
    bat0 = wid * BAT_PER_W

    pltpu.sync_copy(idx_hbm.at[pl.ds(bat0, BAT_PER_W)], idx_v)

    def fire_gathers(slab, h):
        for j in range(NB):
            pltpu.make_async_copy(
                table_hbm.at[idx_v.at[slab * NB + j]],
                rows_v.at[h, j],
                gsem[h],
            ).start()

    def drain_gathers(slab, h):
        for j in range(NB):
            pltpu.make_async_copy(
                table_hbm.at[idx_v.at[slab * NB + j]],
                rows_v.at[h, j],
                gsem[h],
            ).wait()

    def out_desc(slab, h):
        return pltpu.make_async_copy(
            rows_v.at[h],
            out_hbm.at[pl.ds(bat0 + slab * NB, NB)],
            osem[h],
        )

    fire_gathers(0, 0)
    fire_gathers(1, 1)

    def body(p, _):
        for h in range(2):
            slab = 2 * p + h
            drain_gathers(slab, h)
            out_desc(slab, h).start()

            @pl.when(p + 1 < N_PAIRS)
            def _():
                out_desc(slab, h).wait()
                fire_gathers(slab + 2, h)

        return 0

    lax.fori_loop(0, N_PAIRS, body, 0)
    out_desc(NSLAB - 2, 0).wait()
    out_desc(NSLAB - 1, 1).wait()


def kernel(X, W):
    idx = X.astype(jnp.int32)
    return _gather_kernel(idx, W)


# speedup vs baseline: 10.2589x; 1.7245x over previous
import functools

import jax
import jax.numpy as jnp
from jax import lax
from jax.experimental import pallas as pl
from jax.experimental.pallas import tpu as pltpu
from jax.experimental.pallas import tpu_sc as plsc

NBATCH = 4096
SEQ = 50
D = 128

_info = plsc.get_sparse_core_info()
NC, NS = _info.num_cores, _info.num_subcores
NW = NC * NS
BAT_PER_W = NBATCH // NW
NBUF = 5
N_GROUPS = SEQ // NBUF

_mesh = plsc.VectorSubcoreMesh(core_axis_name="c", subcore_axis_name="s")


@functools.partial(
    pl.kernel,
    mesh=_mesh,
    out_type=jax.ShapeDtypeStruct((SEQ, NBATCH, D), jnp.float32),
    scratch_types=[
        pltpu.VMEM((SEQ, BAT_PER_W), jnp.int32),
        pltpu.VMEM((NBUF, BAT_PER_W, D), jnp.float32),
    ]
    + [pltpu.SemaphoreType.DMA] * (2 * NBUF),
)
def _gather_kernel(idx_hbm, table_hbm, out_hbm, idx_v, rows_v, *sems):
    gsem = sems[:NBUF]
    osem = sems[NBUF:]
    wid = lax.axis_index("s") * NC + lax.axis_index("c")
    bat0 = wid * BAT_PER_W

    pltpu.sync_copy(idx_hbm.at[:, pl.ds(bat0, BAT_PER_W)], idx_v)

    def gather_desc(s, b):
        return pltpu.make_async_copy(
            table_hbm.at[idx_v.at[s]], rows_v.at[b], gsem[b]
        )

    def out_desc(s, b):
        return pltpu.make_async_copy(
            rows_v.at[b], out_hbm.at[s, pl.ds(bat0, BAT_PER_W)], osem[b]
        )

    for b in range(NBUF):
        gather_desc(b, b).start()

    def body(g, _):
        for b in range(NBUF):
            s = g * NBUF + b
            gather_desc(s, b).wait()
            out_desc(s, b).start()
        for b in range(NBUF):
            s = g * NBUF + b
            out_desc(s, b).wait()

            @pl.when(g + 1 < N_GROUPS)
            def _():
                gather_desc(s + NBUF, b).start()

        return 0

    lax.fori_loop(0, N_GROUPS, body, 0)


def kernel(X, W):
    idxT = X.T.astype(jnp.int32)
    outT = _gather_kernel(idxT, W)
    return outT.transpose(1, 0, 2)
